# Initial kernel scaffold; baseline (speedup 1.0000x reference)
import jax
import jax.numpy as jnp
from jax.experimental import pallas as pl


def kernel(z, codebook):
    raise NotImplementedError("write your pallas kernel here")


# speedup vs baseline: 9.7329x; 9.7329x over previous
import functools

import jax
import jax.numpy as jnp
from jax import lax
from jax.experimental import pallas as pl
from jax.experimental.pallas import tpu as pltpu
from jax.experimental.pallas import tpu_sc as plsc

_K = 8192
_D = 256
_N = 16384
_BETA = 0.25
_TM = 512
_TK = 1024


def _argmin_body(s_z_ref, z_ref, cb_ref, s_c_ref, idx_ref):
    zb2 = z_ref[...] * jnp.bfloat16(2.0)
    szt = s_z_ref[...]
    BOUNDS = (0, 2736, 5472, _K)
    s_min = [jnp.full((_TM, 1), jnp.inf, jnp.float32) for _ in range(3)]
    s_idx = [jnp.zeros((_TM, 1), jnp.float32) for _ in range(3)]
    io = lax.broadcasted_iota(jnp.int32, (_TM, _TK), 1)
    io_f = io.astype(jnp.float32)
    for j in range(_K // _TK):
        cb = cb_ref[pl.ds(j * _TK, _TK), :]
        mm2 = lax.dot_general(zb2, cb, (((1,), (1,)), ((), ())),
                              preferred_element_type=jnp.float32)
        d = (szt + s_c_ref[:, pl.ds(j * _TK, _TK)]) - mm2
        c0, c1 = j * _TK, (j + 1) * _TK
        for s in range(3):
            lo, hi = max(BOUNDS[s], c0), min(BOUNDS[s + 1], c1)
            if lo >= hi:
                continue
            if lo == c0 and hi == c1:
                dm = d
            else:
                seg = (io >= (lo - c0)) & (io < (hi - c0))
                dm = jnp.where(seg, d, jnp.float32(jnp.inf))
            cmin = jnp.min(dm, axis=1, keepdims=True)
            cidx = jnp.min(jnp.where(dm == cmin, io_f, jnp.float32(3e38)),
                           axis=1, keepdims=True) + jnp.float32(c0)
            upd = cmin < s_min[s]
            s_idx[s] = jnp.where(upd, cidx, s_idx[s])
            s_min[s] = jnp.where(upd, cmin, s_min[s])
    def _rne_bf16(x):
        u = lax.bitcast_convert_type(x, jnp.uint32)
        r = (u + jnp.uint32(0x7FFF) + ((u >> 16) & jnp.uint32(1))) \
            & jnp.uint32(0xFFFF0000)
        return lax.bitcast_convert_type(r, jnp.float32)

    av = jnp.full((_TM, 1), jnp.inf, jnp.float32)
    ai = jnp.zeros((_TM, 1), jnp.float32)
    for s in range(3):
        take = (s_min[s] < av) | ((s_min[s] == av) & (s_idx[s] < ai))
        ai = jnp.where(take, s_idx[s], ai)
        av = jnp.where(take, _rne_bf16(s_min[s]), av)
    idx_ref[...] = ai.astype(jnp.int32)


_argmin_call = pl.pallas_call(
    _argmin_body,
    grid=(_N // _TM,),
    in_specs=[
        pl.BlockSpec((_TM, 1), lambda i: (i, 0)),
        pl.BlockSpec((_TM, _D), lambda i: (i, 0)),
        pl.BlockSpec((_K, _D), lambda i: (0, 0)),
        pl.BlockSpec((1, _K), lambda i: (0, 0)),
    ],
    out_specs=pl.BlockSpec((_TM, 1), lambda i: (i, 0)),
    out_shape=jax.ShapeDtypeStruct((_N, 1), jnp.int32),
)


def _st_loss_body(z_ref, zq_ref, out_ref, loss_ref):
    i = pl.program_id(0)
    zv = z_ref[...]
    t = zq_ref[...] - zv
    out_ref[...] = zv + t

    @pl.when(i == 0)
    def _():
        loss_ref[...] = jnp.zeros((1, 1), jnp.float32)

    loss_ref[...] += jnp.sum(t * t, axis=(0, 1), keepdims=True)


_st_loss_call = pl.pallas_call(
    _st_loss_body,
    grid=(_N // _TM,),
    in_specs=[
        pl.BlockSpec((_TM, _D), lambda i: (i, 0)),
        pl.BlockSpec((_TM, _D), lambda i: (i, 0)),
    ],
    out_specs=[
        pl.BlockSpec((_TM, _D), lambda i: (i, 0)),
        pl.BlockSpec((1, 1), lambda i: (0, 0)),
    ],
    out_shape=[
        jax.ShapeDtypeStruct((_N, _D), jnp.float32),
        jax.ShapeDtypeStruct((1, 1), jnp.float32),
    ],
)


def _make_sc_gather():
    ncores, nsub = 2, 16
    nw = ncores * nsub
    b_per_w = _N // nw
    ch = 128
    mesh = plsc.VectorSubcoreMesh(core_axis_name="c", subcore_axis_name="s")

    @functools.partial(
        pl.kernel, mesh=mesh,
        out_type=jax.ShapeDtypeStruct((_N, _D), jnp.float32),
        scratch_types=[
            pltpu.VMEM((ch,), jnp.int32),
            pltpu.VMEM((ch, _D), jnp.float32),
            pltpu.SemaphoreType.DMA,
        ],
    )
    def gather(table_hbm, idx_hbm, out_hbm, idx_v, rows_v, sem):
        wid = lax.axis_index("s") * ncores + lax.axis_index("c")
        base = wid * b_per_w
        for j in range(b_per_w // ch):
            off = base + j * ch
            pltpu.sync_copy(idx_hbm.at[pl.ds(off, ch)], idx_v)
            pltpu.async_copy(table_hbm.at[idx_v], rows_v, sem).wait()
            pltpu.sync_copy(rows_v, out_hbm.at[pl.ds(off, ch)])

    return gather


@functools.cache
def _sc_gather():
    return _make_sc_gather()


def kernel(z, codebook):
    z2 = z.reshape(-1, _D)
    s_z = jnp.sum(z2 ** 2, axis=1, keepdims=True)
    s_c = jnp.sum(codebook ** 2, axis=1)

    idx2 = _argmin_call(s_z, z2.astype(jnp.bfloat16),
                        codebook.astype(jnp.bfloat16),
                        s_c.reshape(1, _K))
    z_q = _sc_gather()(codebook, idx2.reshape(-1))

    out, loss_acc = _st_loss_call(z2, z_q)
    mean = loss_acc[0, 0] / jnp.float32(_N * _D)
    loss = mean + jnp.float32(_BETA) * mean
    return out.reshape(z.shape), idx2, loss


# speedup vs baseline: 10.6253x; 1.0917x over previous
import functools

import jax
import jax.numpy as jnp
from jax import lax
from jax.experimental import pallas as pl
from jax.experimental.pallas import tpu as pltpu
from jax.experimental.pallas import tpu_sc as plsc

_K = 8192
_D = 256
_N = 16384
_BETA = 0.25
_TM = 1024
_TK = 1024


def _argmin_body(s_z_ref, z_ref, cb_ref, s_c_ref, idx_ref):
    zb2 = z_ref[...] * jnp.bfloat16(2.0)
    szt = s_z_ref[...]
    BOUNDS = (0, 2736, 5472, _K)
    s_min = [jnp.full((_TM, 1), jnp.inf, jnp.float32) for _ in range(3)]
    s_idx = [jnp.zeros((_TM, 1), jnp.float32) for _ in range(3)]
    io = lax.broadcasted_iota(jnp.int32, (_TM, _TK), 1)
    io_f = io.astype(jnp.float32)
    for j in range(_K // _TK):
        cb = cb_ref[pl.ds(j * _TK, _TK), :]
        mm2 = lax.dot_general(zb2, cb, (((1,), (1,)), ((), ())),
                              preferred_element_type=jnp.float32)
        d = (szt + s_c_ref[:, pl.ds(j * _TK, _TK)]) - mm2
        c0, c1 = j * _TK, (j + 1) * _TK
        for s in range(3):
            lo, hi = max(BOUNDS[s], c0), min(BOUNDS[s + 1], c1)
            if lo >= hi:
                continue
            if lo == c0 and hi == c1:
                dm = d
            else:
                seg = (io >= (lo - c0)) & (io < (hi - c0))
                dm = jnp.where(seg, d, jnp.float32(jnp.inf))
            cmin = jnp.min(dm, axis=1, keepdims=True)
            cidx = jnp.min(jnp.where(dm == cmin, io_f, jnp.float32(3e38)),
                           axis=1, keepdims=True) + jnp.float32(c0)
            upd = cmin < s_min[s]
            s_idx[s] = jnp.where(upd, cidx, s_idx[s])
            s_min[s] = jnp.where(upd, cmin, s_min[s])
    def _rne_bf16(x):
        u = lax.bitcast_convert_type(x, jnp.uint32)
        r = (u + jnp.uint32(0x7FFF) + ((u >> 16) & jnp.uint32(1))) \
            & jnp.uint32(0xFFFF0000)
        return lax.bitcast_convert_type(r, jnp.float32)

    av = jnp.full((_TM, 1), jnp.inf, jnp.float32)
    ai = jnp.zeros((_TM, 1), jnp.float32)
    for s in range(3):
        take = (s_min[s] < av) | ((s_min[s] == av) & (s_idx[s] < ai))
        ai = jnp.where(take, s_idx[s], ai)
        av = jnp.where(take, _rne_bf16(s_min[s]), av)
    idx_ref[...] = ai.astype(jnp.int32)


_argmin_call = pl.pallas_call(
    _argmin_body,
    grid=(_N // _TM,),
    in_specs=[
        pl.BlockSpec((_TM, 1), lambda i: (i, 0)),
        pl.BlockSpec((_TM, _D), lambda i: (i, 0)),
        pl.BlockSpec((_K, _D), lambda i: (0, 0)),
        pl.BlockSpec((1, _K), lambda i: (0, 0)),
    ],
    out_specs=pl.BlockSpec((_TM, 1), lambda i: (i, 0)),
    out_shape=jax.ShapeDtypeStruct((_N, 1), jnp.int32),
)


def _st_loss_body(z_ref, zq_ref, out_ref, loss_ref):
    i = pl.program_id(0)
    zv = z_ref[...]
    t = zq_ref[...] - zv
    out_ref[...] = zv + t

    @pl.when(i == 0)
    def _():
        loss_ref[...] = jnp.zeros((1, 1), jnp.float32)

    loss_ref[...] += jnp.sum(t * t, axis=(0, 1), keepdims=True)


_st_loss_call = pl.pallas_call(
    _st_loss_body,
    grid=(_N // _TM,),
    in_specs=[
        pl.BlockSpec((_TM, _D), lambda i: (i, 0)),
        pl.BlockSpec((_TM, _D), lambda i: (i, 0)),
    ],
    out_specs=[
        pl.BlockSpec((_TM, _D), lambda i: (i, 0)),
        pl.BlockSpec((1, 1), lambda i: (0, 0)),
    ],
    out_shape=[
        jax.ShapeDtypeStruct((_N, _D), jnp.float32),
        jax.ShapeDtypeStruct((1, 1), jnp.float32),
    ],
)


def _make_sc_gather():
    ncores, nsub = 2, 16
    nw = ncores * nsub
    b_per_w = _N // nw
    ch = 128
    mesh = plsc.VectorSubcoreMesh(core_axis_name="c", subcore_axis_name="s")

    @functools.partial(
        pl.kernel, mesh=mesh,
        out_type=jax.ShapeDtypeStruct((_N, _D), jnp.float32),
        scratch_types=[
            pltpu.VMEM((ch,), jnp.int32),
            pltpu.VMEM((ch, _D), jnp.float32),
            pltpu.SemaphoreType.DMA,
        ],
    )
    def gather(table_hbm, idx_hbm, out_hbm, idx_v, rows_v, sem):
        wid = lax.axis_index("s") * ncores + lax.axis_index("c")
        base = wid * b_per_w
        for j in range(b_per_w // ch):
            off = base + j * ch
            pltpu.sync_copy(idx_hbm.at[pl.ds(off, ch)], idx_v)
            pltpu.async_copy(table_hbm.at[idx_v], rows_v, sem).wait()
            pltpu.sync_copy(rows_v, out_hbm.at[pl.ds(off, ch)])

    return gather


@functools.cache
def _sc_gather():
    return _make_sc_gather()


def kernel(z, codebook):
    z2 = z.reshape(-1, _D)
    s_z = jnp.sum(z2 ** 2, axis=1, keepdims=True)
    s_c = jnp.sum(codebook ** 2, axis=1)

    idx2 = _argmin_call(s_z, z2.astype(jnp.bfloat16),
                        codebook.astype(jnp.bfloat16),
                        s_c.reshape(1, _K))
    z_q = _sc_gather()(codebook, idx2.reshape(-1))

    out, loss_acc = _st_loss_call(z2, z_q)
    mean = loss_acc[0, 0] / jnp.float32(_N * _D)
    loss = mean + jnp.float32(_BETA) * mean
    return out.reshape(z.shape), idx2, loss


# speedup vs baseline: 11.0260x; 1.0377x over previous
import functools

import jax
import jax.numpy as jnp
from jax import lax
from jax.experimental import pallas as pl
from jax.experimental.pallas import tpu as pltpu
from jax.experimental.pallas import tpu_sc as plsc

_K = 8192
_D = 256
_N = 16384
_BETA = 0.25
_TM = 2048
_TK = 1024


def _argmin_body(s_z_ref, z_ref, cb_ref, s_c_ref, idx_ref):
    zb2 = z_ref[...] * jnp.bfloat16(2.0)
    szt = s_z_ref[...]
    BOUNDS = (0, 2736, 5472, _K)
    s_min = [jnp.full((_TM, 1), jnp.inf, jnp.float32) for _ in range(3)]
    s_idx = [jnp.zeros((_TM, 1), jnp.float32) for _ in range(3)]
    io = lax.broadcasted_iota(jnp.int32, (_TM, _TK), 1)
    io_f = io.astype(jnp.float32)
    for j in range(_K // _TK):
        cb = cb_ref[pl.ds(j * _TK, _TK), :]
        mm2 = lax.dot_general(zb2, cb, (((1,), (1,)), ((), ())),
                              preferred_element_type=jnp.float32)
        d = (szt + s_c_ref[:, pl.ds(j * _TK, _TK)]) - mm2
        c0, c1 = j * _TK, (j + 1) * _TK
        for s in range(3):
            lo, hi = max(BOUNDS[s], c0), min(BOUNDS[s + 1], c1)
            if lo >= hi:
                continue
            if lo == c0 and hi == c1:
                dm = d
            else:
                seg = (io >= (lo - c0)) & (io < (hi - c0))
                dm = jnp.where(seg, d, jnp.float32(jnp.inf))
            cmin = jnp.min(dm, axis=1, keepdims=True)
            cidx = jnp.min(jnp.where(dm == cmin, io_f, jnp.float32(3e38)),
                           axis=1, keepdims=True) + jnp.float32(c0)
            upd = cmin < s_min[s]
            s_idx[s] = jnp.where(upd, cidx, s_idx[s])
            s_min[s] = jnp.where(upd, cmin, s_min[s])
    def _rne_bf16(x):
        u = lax.bitcast_convert_type(x, jnp.uint32)
        r = (u + jnp.uint32(0x7FFF) + ((u >> 16) & jnp.uint32(1))) \
            & jnp.uint32(0xFFFF0000)
        return lax.bitcast_convert_type(r, jnp.float32)

    av = jnp.full((_TM, 1), jnp.inf, jnp.float32)
    ai = jnp.zeros((_TM, 1), jnp.float32)
    for s in range(3):
        take = (s_min[s] < av) | ((s_min[s] == av) & (s_idx[s] < ai))
        ai = jnp.where(take, s_idx[s], ai)
        av = jnp.where(take, _rne_bf16(s_min[s]), av)
    idx_ref[...] = ai.astype(jnp.int32)


_argmin_call = pl.pallas_call(
    _argmin_body,
    grid=(_N // _TM,),
    in_specs=[
        pl.BlockSpec((_TM, 1), lambda i: (i, 0)),
        pl.BlockSpec((_TM, _D), lambda i: (i, 0)),
        pl.BlockSpec((_K, _D), lambda i: (0, 0)),
        pl.BlockSpec((1, _K), lambda i: (0, 0)),
    ],
    out_specs=pl.BlockSpec((_TM, 1), lambda i: (i, 0)),
    out_shape=jax.ShapeDtypeStruct((_N, 1), jnp.int32),
)


def _st_loss_body(z_ref, zq_ref, out_ref, loss_ref):
    i = pl.program_id(0)
    zv = z_ref[...]
    t = zq_ref[...] - zv
    out_ref[...] = zv + t

    @pl.when(i == 0)
    def _():
        loss_ref[...] = jnp.zeros((1, 1), jnp.float32)

    loss_ref[...] += jnp.sum(t * t, axis=(0, 1), keepdims=True)


_st_loss_call = pl.pallas_call(
    _st_loss_body,
    grid=(_N // _TM,),
    in_specs=[
        pl.BlockSpec((_TM, _D), lambda i: (i, 0)),
        pl.BlockSpec((_TM, _D), lambda i: (i, 0)),
    ],
    out_specs=[
        pl.BlockSpec((_TM, _D), lambda i: (i, 0)),
        pl.BlockSpec((1, 1), lambda i: (0, 0)),
    ],
    out_shape=[
        jax.ShapeDtypeStruct((_N, _D), jnp.float32),
        jax.ShapeDtypeStruct((1, 1), jnp.float32),
    ],
)


def _make_sc_gather():
    ncores, nsub = 2, 16
    nw = ncores * nsub
    b_per_w = _N // nw
    ch = 128
    mesh = plsc.VectorSubcoreMesh(core_axis_name="c", subcore_axis_name="s")

    @functools.partial(
        pl.kernel, mesh=mesh,
        out_type=jax.ShapeDtypeStruct((_N, _D), jnp.float32),
        scratch_types=[
            pltpu.VMEM((ch,), jnp.int32),
            pltpu.VMEM((ch, _D), jnp.float32),
            pltpu.SemaphoreType.DMA,
        ],
    )
    def gather(table_hbm, idx_hbm, out_hbm, idx_v, rows_v, sem):
        wid = lax.axis_index("s") * ncores + lax.axis_index("c")
        base = wid * b_per_w
        for j in range(b_per_w // ch):
            off = base + j * ch
            pltpu.sync_copy(idx_hbm.at[pl.ds(off, ch)], idx_v)
            pltpu.async_copy(table_hbm.at[idx_v], rows_v, sem).wait()
            pltpu.sync_copy(rows_v, out_hbm.at[pl.ds(off, ch)])

    return gather


@functools.cache
def _sc_gather():
    return _make_sc_gather()


def kernel(z, codebook):
    z2 = z.reshape(-1, _D)
    s_z = jnp.sum(z2 ** 2, axis=1, keepdims=True)
    s_c = jnp.sum(codebook ** 2, axis=1)

    idx2 = _argmin_call(s_z, z2.astype(jnp.bfloat16),
                        codebook.astype(jnp.bfloat16),
                        s_c.reshape(1, _K))
    z_q = _sc_gather()(codebook, idx2.reshape(-1))

    out, loss_acc = _st_loss_call(z2, z_q)
    mean = loss_acc[0, 0] / jnp.float32(_N * _D)
    loss = mean + jnp.float32(_BETA) * mean
    return out.reshape(z.shape), idx2, loss
